# baseline (speedup 1.0000x reference)
import functools

import jax
import jax.numpy as jnp
from jax import lax
from jax.experimental import pallas as pl
from jax.experimental.pallas import tpu as pltpu
from jax.experimental.pallas import tpu_sc as plsc

B = 4
S = 2048
H = 1024
L = 16
NC = 2
NS = 16
NW = NC * NS
P_PER_W = S // NW
CHUNK = 16
N_CC = P_PER_W // CHUNK
N_CHUNKS = N_CC * B
HK = H // L
NG = 4


def _emb_body(idx_hbm, types_hbm, vocab_hbm, pos_hbm, tt_hbm, out_hbm,
              idx_v, ti_v, tf_v, d_v, pbuf, gbuf,
              gsem0, gsem1, gsem2, gsem3, stsem0, stsem1, stsem2, stsem3,
              psem0, psem1, ssem):
    c = lax.axis_index("c")
    s = lax.axis_index("s")
    wid = s * NC + c
    p0 = wid * P_PER_W
    gsem = (gsem0, gsem1, gsem2, gsem3)
    stsem = (stsem0, stsem1, stsem2, stsem3)
    psem = (psem0, psem1)

    for b in range(2):
        pltpu.async_copy(idx_hbm.at[b, pl.ds(p0, P_PER_W)], idx_v.at[b], ssem)
    for b in range(2):
        pltpu.make_async_copy(
            idx_hbm.at[0, pl.ds(p0, P_PER_W)], idx_v.at[0], ssem).wait()
    for b in range(2, B):
        pltpu.async_copy(idx_hbm.at[b, pl.ds(p0, P_PER_W)], idx_v.at[b], ssem)
    for b in range(B):
        pltpu.async_copy(types_hbm.at[b, pl.ds(p0, P_PER_W)], ti_v.at[b],
                         ssem)
    tt_cp = pltpu.async_copy(tt_hbm, d_v, ssem)

    def issue_pos(cc, slot):
        return pltpu.async_copy(
            pos_hbm.at[pl.ds(p0 + cc * CHUNK, CHUNK)], pbuf.at[slot],
            psem[slot])

    def issue_gather(ch, slot):
        b, cc = ch % B, ch // B
        return pltpu.async_copy(
            vocab_hbm.at[idx_v.at[b, pl.ds(cc * CHUNK, CHUNK)]],
            gbuf.at[slot], gsem[slot])

    issue_pos(0, 0)
    issue_pos(1, 1)
    issue_gather(0, 0)
    issue_gather(1, 1)

    for b in range(2, B):
        pltpu.make_async_copy(
            idx_hbm.at[0, pl.ds(p0, P_PER_W)], idx_v.at[0], ssem).wait()
    for b in range(B):
        pltpu.make_async_copy(
            types_hbm.at[0, pl.ds(p0, P_PER_W)], ti_v.at[0], ssem).wait()
    tt_cp.wait()

    def conv_types(i, carry):
        r = i // (P_PER_W // L)
        kk = i % (P_PER_W // L)
        tf_v[r, pl.ds(kk * L, L)] = (
            ti_v[r, pl.ds(kk * L, L)].astype(jnp.float32))
        return carry
    lax.fori_loop(0, B * P_PER_W // L, conv_types, 0)

    def mk_d(k, carry):
        t0 = d_v[0, pl.ds(k * L, L)]
        t1 = d_v[1, pl.ds(k * L, L)]
        d_v[1, pl.ds(k * L, L)] = t1 - t0
        return carry
    lax.fori_loop(0, HK, mk_d, 0)

    for ch in range(N_CHUNKS):
        cc, b = ch // B, ch % B
        gs = ch % NG
        ps = cc % 2
        if ch + 2 < N_CHUNKS:
            ns = (ch + 2) % NG
            if ch >= 2:
                pltpu.make_async_copy(
                    gbuf.at[ns], out_hbm.at[0, pl.ds(0, CHUNK)],
                    stsem[ns]).wait()
            issue_gather(ch + 2, ns)
        if b == 0:
            pltpu.make_async_copy(
                pos_hbm.at[pl.ds(p0, CHUNK)], pbuf.at[ps], psem[ps]).wait()
        pltpu.make_async_copy(
            vocab_hbm.at[idx_v.at[0, pl.ds(0, CHUNK)]], gbuf.at[gs],
            gsem[gs]).wait()

        tfc = tf_v[b, pl.ds(cc * CHUNK, CHUNK)]
        tj = [tfc.at[jnp.full((L,), j, jnp.int32)].get(
                  mode="promise_in_bounds") for j in range(CHUNK)]

        def do_k(k2, carry2, gs=gs, ps=ps, tj=tj):
            for u in range(2):
                ks = pl.ds((2 * k2 + u) * L, L)
                t0k = d_v[0, ks]
                dk = d_v[1, ks]
                for j in range(CHUNK):
                    p = pbuf[ps, j, ks]
                    plsc.addupdate(gbuf.at[gs, j, ks],
                                   (p + t0k) + tj[j] * dk)
            return carry2
        lax.fori_loop(0, HK // 2, do_k, 0)

        pltpu.async_copy(
            gbuf.at[gs], out_hbm.at[b, pl.ds(p0 + cc * CHUNK, CHUNK)],
            stsem[gs])
        if b == B - 1 and cc + 2 < N_CC:
            issue_pos(cc + 2, ps)

    for ch in (N_CHUNKS - 2, N_CHUNKS - 1):
        pltpu.make_async_copy(
            gbuf.at[ch % NG], out_hbm.at[0, pl.ds(0, CHUNK)],
            stsem[ch % NG]).wait()


@jax.jit
def _emb(idx, types, vocab_weight, pos_weight, tokentype_weight):
    mesh = plsc.VectorSubcoreMesh(core_axis_name="c", subcore_axis_name="s")
    f = functools.partial(
        pl.kernel,
        mesh=mesh,
        out_type=jax.ShapeDtypeStruct((B, S, H), jnp.float32),
        scratch_types=[
            pltpu.VMEM((B, P_PER_W), jnp.int32),
            pltpu.VMEM((B, P_PER_W), jnp.int32),
            pltpu.VMEM((B, P_PER_W), jnp.float32),
            pltpu.VMEM((2, H), jnp.float32),
            pltpu.VMEM((2, CHUNK, H), jnp.float32),
            pltpu.VMEM((NG, CHUNK, H), jnp.float32),
            pltpu.SemaphoreType.DMA,
            pltpu.SemaphoreType.DMA,
            pltpu.SemaphoreType.DMA,
            pltpu.SemaphoreType.DMA,
            pltpu.SemaphoreType.DMA,
            pltpu.SemaphoreType.DMA,
            pltpu.SemaphoreType.DMA,
            pltpu.SemaphoreType.DMA,
            pltpu.SemaphoreType.DMA,
            pltpu.SemaphoreType.DMA,
            pltpu.SemaphoreType.DMA,
        ],
    )(_emb_body)
    return f(idx, types, vocab_weight, pos_weight, tokentype_weight)


def kernel(idx, types, vocab_weight, pos_weight, tokentype_weight):
    return _emb(idx, types, vocab_weight, pos_weight, tokentype_weight)


# speedup vs baseline: 2.0654x; 2.0654x over previous
import functools

import jax
import jax.numpy as jnp
from jax import lax
from jax.experimental import pallas as pl
from jax.experimental.pallas import tpu as pltpu
from jax.experimental.pallas import tpu_sc as plsc

B = 4
S = 2048
H = 1024
L = 16
NC = 2
NS = 16
NW = NC * NS
P_PER_W = S // NW
CHUNK = 16
N_CC = P_PER_W // CHUNK
N_CHUNKS = N_CC * B
HK = H // L
NG = 4


def _emb_body(idx_hbm, types_hbm, vocab_hbm, pos_hbm, tt_hbm, out_hbm,
              idx_v, ti_v, tf_v, d_v, pbuf, gbuf,
              gsem0, gsem1, gsem2, gsem3, stsem0, stsem1, stsem2, stsem3,
              psem0, psem1, ssem):
    c = lax.axis_index("c")
    s = lax.axis_index("s")
    wid = s * NC + c
    p0 = wid * P_PER_W
    gsem = (gsem0, gsem1, gsem2, gsem3)
    stsem = (stsem0, stsem1, stsem2, stsem3)
    psem = (psem0, psem1)

    for b in range(2):
        pltpu.async_copy(idx_hbm.at[b, pl.ds(p0, P_PER_W)], idx_v.at[b], ssem)
    for b in range(2):
        pltpu.make_async_copy(
            idx_hbm.at[0, pl.ds(p0, P_PER_W)], idx_v.at[0], ssem).wait()
    for b in range(2, B):
        pltpu.async_copy(idx_hbm.at[b, pl.ds(p0, P_PER_W)], idx_v.at[b], ssem)
    for b in range(B):
        pltpu.async_copy(types_hbm.at[b, pl.ds(p0, P_PER_W)], ti_v.at[b],
                         ssem)
    tt_cp = pltpu.async_copy(tt_hbm, d_v, ssem)

    def issue_pos(cc, slot):
        return pltpu.async_copy(
            pos_hbm.at[pl.ds(p0 + cc * CHUNK, CHUNK)], pbuf.at[slot],
            psem[slot])

    def issue_gather(ch, slot):
        b, cc = ch % B, ch // B
        return pltpu.async_copy(
            vocab_hbm.at[idx_v.at[b, pl.ds(cc * CHUNK, CHUNK)]],
            gbuf.at[slot], gsem[slot])

    issue_pos(0, 0)
    issue_pos(1, 1)
    issue_gather(0, 0)
    issue_gather(1, 1)

    for b in range(2, B):
        pltpu.make_async_copy(
            idx_hbm.at[0, pl.ds(p0, P_PER_W)], idx_v.at[0], ssem).wait()
    for b in range(B):
        pltpu.make_async_copy(
            types_hbm.at[0, pl.ds(p0, P_PER_W)], ti_v.at[0], ssem).wait()
    tt_cp.wait()

    def conv_types(i, carry):
        r = i // (P_PER_W // L)
        kk = i % (P_PER_W // L)
        tf_v[r, pl.ds(kk * L, L)] = (
            ti_v[r, pl.ds(kk * L, L)].astype(jnp.float32))
        return carry
    lax.fori_loop(0, B * P_PER_W // L, conv_types, 0)

    def mk_d(k, carry):
        t0 = d_v[0, pl.ds(k * L, L)]
        t1 = d_v[1, pl.ds(k * L, L)]
        d_v[1, pl.ds(k * L, L)] = t1 - t0
        return carry
    lax.fori_loop(0, HK, mk_d, 0)

    for ch in range(N_CHUNKS):
        cc, b = ch // B, ch % B
        gs = ch % NG
        ps = cc % 2
        if ch + 2 < N_CHUNKS:
            ns = (ch + 2) % NG
            if ch >= 2:
                pltpu.make_async_copy(
                    gbuf.at[ns], out_hbm.at[0, pl.ds(0, CHUNK)],
                    stsem[ns]).wait()
            issue_gather(ch + 2, ns)
        if b == 0:
            pltpu.make_async_copy(
                pos_hbm.at[pl.ds(p0, CHUNK)], pbuf.at[ps], psem[ps]).wait()
        pltpu.make_async_copy(
            vocab_hbm.at[idx_v.at[0, pl.ds(0, CHUNK)]], gbuf.at[gs],
            gsem[gs]).wait()

        tfc = tf_v[b, pl.ds(cc * CHUNK, CHUNK)]
        tj = [tfc.at[jnp.full((L,), j, jnp.int32)].get(
                  mode="promise_in_bounds") for j in range(CHUNK)]

        def do_k(k, carry2, gs=gs, ps=ps, tj=tj):
            ks = pl.ds(k * L, L)
            t0k = d_v[0, ks]
            dk = d_v[1, ks]
            for j in range(CHUNK):
                p = pbuf[ps, j, ks]
                plsc.addupdate(gbuf.at[gs, j, ks], (p + t0k) + tj[j] * dk)
            return carry2
        lax.fori_loop(0, HK, do_k, 0)

        pltpu.async_copy(
            gbuf.at[gs], out_hbm.at[b, pl.ds(p0 + cc * CHUNK, CHUNK)],
            stsem[gs])
        if b == B - 1 and cc + 2 < N_CC:
            issue_pos(cc + 2, ps)

    for ch in (N_CHUNKS - 2, N_CHUNKS - 1):
        pltpu.make_async_copy(
            gbuf.at[ch % NG], out_hbm.at[0, pl.ds(0, CHUNK)],
            stsem[ch % NG]).wait()


@jax.jit
def _emb(idx, types, vocab_weight, pos_weight, tokentype_weight):
    mesh = plsc.VectorSubcoreMesh(core_axis_name="c", subcore_axis_name="s")
    f = functools.partial(
        pl.kernel,
        mesh=mesh,
        out_type=jax.ShapeDtypeStruct((B, S, H), jnp.float32),
        scratch_types=[
            pltpu.VMEM((B, P_PER_W), jnp.int32),
            pltpu.VMEM((B, P_PER_W), jnp.int32),
            pltpu.VMEM((B, P_PER_W), jnp.float32),
            pltpu.VMEM((2, H), jnp.float32),
            pltpu.VMEM((2, CHUNK, H), jnp.float32),
            pltpu.VMEM((NG, CHUNK, H), jnp.float32),
            pltpu.SemaphoreType.DMA,
            pltpu.SemaphoreType.DMA,
            pltpu.SemaphoreType.DMA,
            pltpu.SemaphoreType.DMA,
            pltpu.SemaphoreType.DMA,
            pltpu.SemaphoreType.DMA,
            pltpu.SemaphoreType.DMA,
            pltpu.SemaphoreType.DMA,
            pltpu.SemaphoreType.DMA,
            pltpu.SemaphoreType.DMA,
            pltpu.SemaphoreType.DMA,
        ],
    )(_emb_body)
    return f(idx, types, vocab_weight, pos_weight, tokentype_weight)


def kernel(idx, types, vocab_weight, pos_weight, tokentype_weight):
    return _emb(idx, types, vocab_weight, pos_weight, tokentype_weight)
